# baseline (speedup 1.0000x reference)
import functools

import jax
import jax.numpy as jnp
from jax import lax
from jax.experimental import pallas as pl
from jax.experimental.pallas import tpu as pltpu
from jax.experimental.pallas import tpu_sc as plsc

B = 1024
IN = 768
H = 512
BLK = 256

try:
    _SC_INFO = plsc.get_sparse_core_info()
    _NC = _SC_INFO.num_cores
    _NS = _SC_INFO.num_subcores
except Exception:
    _NC, _NS = 2, 16
_NC = 1
_NW = _NC * _NS
_BPW = B // _NW


def _vqvae_body(x_ref, w1_ref, b1_ref, w2_ref, b2_ref, cb_ref, w3_ref,
                b3_ref, w4_ref, b4_ref, xr_ref, ze_ref, idx_ref):
    x = x_ref[...]
    h = jnp.maximum(
        jnp.dot(x, w1_ref[...], preferred_element_type=jnp.float32) + b1_ref[...], 0.0)
    z_e = jnp.maximum(
        jnp.dot(h, w2_ref[...], preferred_element_type=jnp.float32) + b2_ref[...], 0.0)

    cb = cb_ref[...]
    rs = jnp.sum(cb, axis=1)[None, :]
    q = jnp.sum(cb * cb, axis=1)[None, :]
    scores = jnp.float32(H) * z_e * z_e - 2.0 * z_e * rs + q

    mn = jnp.min(scores, axis=1, keepdims=True)
    iota = lax.broadcasted_iota(jnp.int32, scores.shape, 1)
    idx = jnp.min(jnp.where(scores == mn, iota, H), axis=1, keepdims=True)
    onehot = (iota == idx).astype(jnp.float32)
    z_q = jnp.dot(onehot, cb, preferred_element_type=jnp.float32)

    d = jnp.maximum(
        jnp.dot(z_q, w3_ref[...], preferred_element_type=jnp.float32) + b3_ref[...], 0.0)
    logits = jnp.dot(d, w4_ref[...], preferred_element_type=jnp.float32) + b4_ref[...]
    xr_ref[...] = jax.nn.sigmoid(logits)
    ze_ref[...] = z_e

    ir = lax.broadcasted_iota(jnp.int32, (BLK, BLK), 0)
    ic = lax.broadcasted_iota(jnp.int32, (BLK, BLK), 1)
    eye = (ir == ic).astype(jnp.float32)
    diag = eye * idx.astype(jnp.float32)
    row = jnp.dot(jnp.ones((1, BLK), jnp.float32), diag,
                  preferred_element_type=jnp.float32,
                  precision=lax.Precision.HIGHEST)
    idx_ref[...] = row.astype(jnp.int32).reshape(BLK)


def _gather_sc_body(idx_hbm, cb_hbm, zq_hbm, idx_v, rows_v, sem):
    wid = lax.axis_index("s") * _NC + lax.axis_index("c")
    base = wid * _BPW
    pltpu.sync_copy(idx_hbm.at[pl.ds(base, _BPW)], idx_v)
    pltpu.async_copy(cb_hbm.at[idx_v], rows_v, sem).wait()
    pltpu.sync_copy(rows_v, zq_hbm.at[pl.ds(base, _BPW)])


def _tc_fused(x, W1, b1, W2, b2, codebook, W3, b3, W4, b4):
    grid = (B // BLK,)
    full = lambda shape: pl.BlockSpec(shape, lambda i: (0, 0))
    row_blk = lambda cols: pl.BlockSpec((BLK, cols), lambda i: (i, 0))
    return pl.pallas_call(
        _vqvae_body,
        grid=grid,
        in_specs=[
            row_blk(IN),
            full((IN, H)), full((1, H)),
            full((H, H)), full((1, H)),
            full((H, H)),
            full((H, H)), full((1, H)),
            full((H, IN)), full((1, IN)),
        ],
        out_specs=[row_blk(IN), row_blk(H),
                   pl.BlockSpec((BLK,), lambda i: (i,))],
        out_shape=[
            jax.ShapeDtypeStruct((B, IN), jnp.float32),
            jax.ShapeDtypeStruct((B, H), jnp.float32),
            jax.ShapeDtypeStruct((B,), jnp.int32),
        ],
        compiler_params=pltpu.CompilerParams(
            dimension_semantics=("arbitrary",)),
    )(x, W1, b1.reshape(1, H), W2, b2.reshape(1, H), codebook,
      W3, b3.reshape(1, H), W4, b4.reshape(1, IN))


@functools.cache
def _sc_gather():
    return functools.partial(
        pl.kernel,
        out_type=jax.ShapeDtypeStruct((B, H), jnp.float32),
        mesh=plsc.VectorSubcoreMesh(core_axis_name="c", subcore_axis_name="s",
                                    num_cores=_NC),
        compiler_params=pltpu.CompilerParams(use_tc_tiling_on_sc=False),
        scratch_types=[
            pltpu.VMEM((_BPW,), jnp.int32),
            pltpu.VMEM((_BPW, H), jnp.float32),
            pltpu.SemaphoreType.DMA,
        ],
    )(_gather_sc_body)


@jax.jit
def kernel(x, W1, b1, W2, b2, codebook, W3, b3, W4, b4):
    x_recon, z_e, idx = _tc_fused(x, W1, b1, W2, b2, codebook, W3, b3, W4, b4)
    z_q = _sc_gather()(idx, codebook)
    return (x_recon, z_e, z_q)


# speedup vs baseline: 1.0948x; 1.0948x over previous
import functools

import jax
import jax.numpy as jnp
from jax import lax
from jax.experimental import pallas as pl
from jax.experimental.pallas import tpu as pltpu
from jax.experimental.pallas import tpu_sc as plsc

B = 1024
IN = 768
H = 512
BLK = 256

try:
    _SC_INFO = plsc.get_sparse_core_info()
    _NC = _SC_INFO.num_cores
    _NS = _SC_INFO.num_subcores
except Exception:
    _NC, _NS = 2, 16
_NC = 1
_NW = _NC * _NS
_BPW = B // _NW


def _vqvae_body(x_ref, w1_ref, b1_ref, w2_ref, b2_ref, cb_ref, w3_ref,
                b3_ref, w4_ref, b4_ref, xr_ref, ze_ref, idx_ref):
    x = x_ref[...]
    h = jnp.maximum(
        jnp.dot(x, w1_ref[...], preferred_element_type=jnp.float32) + b1_ref[...], 0.0)
    z_e = jnp.maximum(
        jnp.dot(h, w2_ref[...], preferred_element_type=jnp.float32) + b2_ref[...], 0.0)

    cb = cb_ref[...]
    rs = jnp.sum(cb, axis=1)[None, :]
    q = jnp.sum(cb * cb, axis=1)[None, :]
    scores = jnp.float32(H) * z_e * z_e - 2.0 * z_e * rs + q

    mn = jnp.min(scores, axis=1, keepdims=True)
    iota = lax.broadcasted_iota(jnp.int32, scores.shape, 1)
    idx = jnp.min(jnp.where(scores == mn, iota, H), axis=1, keepdims=True)
    onehot = (iota == idx).astype(jnp.float32)
    z_q = jnp.dot(onehot, cb, preferred_element_type=jnp.float32)

    d = jnp.maximum(
        jnp.dot(z_q, w3_ref[...], preferred_element_type=jnp.float32) + b3_ref[...], 0.0)
    logits = jnp.dot(d, w4_ref[...], preferred_element_type=jnp.float32) + b4_ref[...]
    xr_ref[...] = jax.nn.sigmoid(logits)
    ze_ref[...] = z_e

    ir = lax.broadcasted_iota(jnp.int32, (BLK, BLK), 0)
    ic = lax.broadcasted_iota(jnp.int32, (BLK, BLK), 1)
    eye = (ir == ic).astype(jnp.float32)
    diag = eye * idx.astype(jnp.float32)
    row = jnp.dot(jnp.ones((1, BLK), jnp.float32), diag,
                  preferred_element_type=jnp.float32,
                  precision=lax.Precision.HIGHEST)
    idx_ref[...] = row.astype(jnp.int32).reshape(BLK)


def _gather_sc_body(idx_hbm, cb_hbm, zq_hbm, idx_v, rows_v, sem):
    wid = lax.axis_index("s") * _NC + lax.axis_index("c")
    base = wid * _BPW
    pltpu.sync_copy(idx_hbm.at[pl.ds(base, _BPW)], idx_v)
    pltpu.async_copy(cb_hbm.at[idx_v], rows_v, sem).wait()
    pltpu.sync_copy(rows_v, zq_hbm.at[pl.ds(base, _BPW)])


def _tc_fused(x, W1, b1, W2, b2, codebook, W3, b3, W4, b4):
    grid = (B // BLK,)
    full = lambda shape: pl.BlockSpec(shape, lambda i: (0, 0))
    row_blk = lambda cols: pl.BlockSpec((BLK, cols), lambda i: (i, 0))
    return pl.pallas_call(
        _vqvae_body,
        grid=grid,
        in_specs=[
            row_blk(IN),
            full((IN, H)), full((1, H)),
            full((H, H)), full((1, H)),
            full((H, H)),
            full((H, H)), full((1, H)),
            full((H, IN)), full((1, IN)),
        ],
        out_specs=[row_blk(IN), row_blk(H),
                   pl.BlockSpec((BLK,), lambda i: (i,))],
        out_shape=[
            jax.ShapeDtypeStruct((B, IN), jnp.float32),
            jax.ShapeDtypeStruct((B, H), jnp.float32),
            jax.ShapeDtypeStruct((B,), jnp.int32),
        ],
        compiler_params=pltpu.CompilerParams(
            dimension_semantics=("arbitrary",)),
    )(x, W1, b1.reshape(1, H), W2, b2.reshape(1, H), codebook,
      W3, b3.reshape(1, H), W4, b4.reshape(1, IN))


@functools.cache
def _sc_gather():
    return functools.partial(
        pl.kernel,
        out_type=jax.ShapeDtypeStruct((B, H), jnp.float32),
        mesh=plsc.VectorSubcoreMesh(core_axis_name="c", subcore_axis_name="s",
                                    num_cores=_NC),
        scratch_types=[
            pltpu.VMEM((_BPW,), jnp.int32),
            pltpu.VMEM((_BPW, H), jnp.float32),
            pltpu.SemaphoreType.DMA,
        ],
    )(_gather_sc_body)


@jax.jit
def kernel(x, W1, b1, W2, b2, codebook, W3, b3, W4, b4):
    x_recon, z_e, idx = _tc_fused(x, W1, b1, W2, b2, codebook, W3, b3, W4, b4)
    z_q = _sc_gather()(idx, codebook)
    return (x_recon, z_e, z_q)
